# Initial kernel scaffold; baseline (speedup 1.0000x reference)
import jax
import jax.numpy as jnp
from jax.experimental import pallas as pl


def kernel(x, Wr, br, W1, b1, W2, b2, Wv, bv, Wo, bo, bk_scale):
    raise NotImplementedError("write your pallas kernel here")


# speedup vs baseline: 22.6567x; 22.6567x over previous
import jax
import jax.numpy as jnp
from jax.experimental import pallas as pl
from jax.experimental.pallas import tpu as pltpu

N = 2048
D = 768
E = 8
DFF = 3072
FB = 512
NFB = DFF // FB
CH = 16
NCH = N // CH
V_MAX = 3.0
FEAT_CLAMP = 10.0
NEG_BIG = -1e30


def _ffn_body(x_ref, wr_ref, br_ref, w1_ref, b1_ref, w2_ref, b2_ref,
              wv_ref, bv_ref, out_ref, v_ref, wcol_ref, wall_ref):
    e = pl.program_id(0)
    fb = pl.program_id(1)

    @pl.when((e == 0) & (fb == 0))
    def _router():
        x = x_ref[...]
        logits = jnp.dot(x, wr_ref[...], preferred_element_type=jnp.float32)
        logits = logits + br_ref[...]
        idx = jax.lax.broadcasted_iota(jnp.int32, (N, E), 1)
        m1 = jnp.max(logits, axis=1, keepdims=True)
        i1 = jnp.min(jnp.where(logits == m1, idx, E), axis=1, keepdims=True)
        l2 = jnp.where(idx == i1, NEG_BIG, logits)
        m2 = jnp.max(l2, axis=1, keepdims=True)
        i2 = jnp.min(jnp.where(l2 == m2, idx, E), axis=1, keepdims=True)
        e2 = jnp.exp(m2 - m1)
        denom = 1.0 + e2
        g1 = 1.0 / denom
        g2 = e2 / denom
        wall_ref[...] = (jnp.where(idx == i1, g1, 0.0)
                         + jnp.where(idx == i2, g2, 0.0))
        out_ref[...] = jnp.zeros_like(out_ref)

    @pl.when(fb == 0)
    def _wcol():
        idx = jax.lax.broadcasted_iota(jnp.int32, (N, E), 1)
        sel = jnp.where(idx == e, wall_ref[...], 0.0)
        wcol_ref[...] = jnp.sum(sel, axis=1, keepdims=True)

    w = wcol_ref[...]
    h = jnp.dot(x_ref[...], w1_ref[0], preferred_element_type=jnp.float32)
    h = jnp.maximum(h + b1_ref[0], 0.0) * w
    out_ref[...] += jnp.dot(h, w2_ref[0], preferred_element_type=jnp.float32)

    @pl.when(fb == 0)
    def _bias2():
        out_ref[...] += w * b2_ref[0]

    @pl.when((e == E - 1) & (fb == NFB - 1))
    def _value_head():
        v = jnp.dot(out_ref[...], wv_ref[...],
                    preferred_element_type=jnp.float32)
        v_ref[...] = jnp.clip(v + bv_ref[0, 0], -V_MAX, V_MAX)


def _moe_ffn(x2, Wr, br, W1, b1, W2, b2, Wv, bv):
    return pl.pallas_call(
        _ffn_body,
        grid=(E, NFB),
        in_specs=[
            pl.BlockSpec((N, D), lambda e, f: (0, 0)),
            pl.BlockSpec((D, E), lambda e, f: (0, 0)),
            pl.BlockSpec((1, E), lambda e, f: (0, 0)),
            pl.BlockSpec((1, D, FB), lambda e, f: (e, 0, f)),
            pl.BlockSpec((1, 1, FB), lambda e, f: (e, 0, f)),
            pl.BlockSpec((1, FB, D), lambda e, f: (e, f, 0)),
            pl.BlockSpec((1, 1, D), lambda e, f: (e, 0, 0)),
            pl.BlockSpec((D, 1), lambda e, f: (0, 0)),
            pl.BlockSpec(memory_space=pltpu.SMEM),
        ],
        out_specs=[
            pl.BlockSpec((N, D), lambda e, f: (0, 0)),
            pl.BlockSpec((N, 1), lambda e, f: (0, 0)),
        ],
        out_shape=[
            jax.ShapeDtypeStruct((N, D), jnp.float32),
            jax.ShapeDtypeStruct((N, 1), jnp.float32),
        ],
        scratch_shapes=[
            pltpu.VMEM((N, 1), jnp.float32),
            pltpu.VMEM((N, E), jnp.float32),
        ],
    )(x2, Wr, br.reshape(1, E), W1, b1.reshape(E, 1, DFF), W2,
      b2.reshape(E, 1, D), Wv, bv.reshape(1, 1))


def _directional_scan(dr, mat_ref, start_ref, out_re_ref, out_im_ref):
    one = jnp.ones((NCH, 1), jnp.float32)
    zero = jnp.zeros((NCH, 1), jnp.float32)
    m00r, m00i = one, zero
    m01r, m01i = zero, zero
    m10r, m10i = zero, zero
    m11r, m11i = one, zero
    for j in range(CH):
        dj = dr[:, j:j + 1]
        n10r = dj * m10r - m10i - m00r
        n10i = dj * m10i + m10r - m00i
        n11r = dj * m11r - m11i - m01r
        n11i = dj * m11i + m11r - m01i
        m00r, m00i = m10r, m10i
        m01r, m01i = m11r, m11i
        m10r, m10i = n10r, n10i
        m11r, m11i = n11r, n11i
    mat_ref[:, 0:1] = m00r
    mat_ref[:, 1:2] = m00i
    mat_ref[:, 2:3] = m01r
    mat_ref[:, 3:4] = m01i
    mat_ref[:, 4:5] = m10r
    mat_ref[:, 5:6] = m10i
    mat_ref[:, 6:7] = m11r
    mat_ref[:, 7:8] = m11i

    def boundary(c, carry):
        are, aim = carry
        start_ref[pl.ds(c, 1), :] = jnp.concatenate([are, aim], axis=1)
        row = mat_ref[pl.ds(c, 1), :]
        numr = row[:, 0:1] * are - row[:, 1:2] * aim + row[:, 2:3]
        numi = row[:, 0:1] * aim + row[:, 1:2] * are + row[:, 3:4]
        denr = row[:, 4:5] * are - row[:, 5:6] * aim + row[:, 6:7]
        deni = row[:, 4:5] * aim + row[:, 5:6] * are + row[:, 7:8]
        nrm = denr * denr + deni * deni
        return ((numr * denr + numi * deni) / nrm,
                (numi * denr - numr * deni) / nrm)

    z11 = jnp.zeros((1, 1), jnp.float32)
    jax.lax.fori_loop(0, NCH, boundary, (z11, z11))

    are = start_ref[:, 0:1]
    aim = start_ref[:, 1:2]
    for j in range(CH):
        out_re_ref[:, j:j + 1] = are
        out_im_ref[:, j:j + 1] = aim
        x = dr[:, j:j + 1] - are
        y = 1.0 - aim
        nrm = x * x + y * y
        are = x / nrm
        aim = -y / nrm


def _scan_body(v_ref, vrev_ref, are_ref, aim_ref, bre_ref, bim_ref,
               mat_ref, start_ref):
    _directional_scan(2.0 - v_ref[...], mat_ref, start_ref, are_ref, aim_ref)
    _directional_scan(2.0 - vrev_ref[...], mat_ref, start_ref,
                      bre_ref, bim_ref)


def _bk_scan(v16, v16rev):
    return pl.pallas_call(
        _scan_body,
        grid=(1,),
        in_specs=[
            pl.BlockSpec((NCH, CH), lambda i: (0, 0)),
            pl.BlockSpec((NCH, CH), lambda i: (0, 0)),
        ],
        out_specs=[pl.BlockSpec((NCH, CH), lambda i: (0, 0))] * 4,
        out_shape=[jax.ShapeDtypeStruct((NCH, CH), jnp.float32)] * 4,
        scratch_shapes=[
            pltpu.VMEM((NCH, 8), jnp.float32),
            pltpu.VMEM((NCH, 2), jnp.float32),
        ],
    )(v16, v16rev)


def _combine_body(ffn_ref, v_ref, are_ref, aim_ref, rre_ref, rim_ref,
                  wo_ref, bo_ref, sc_ref, out_ref):
    x = (2.0 - v_ref[...]) - are_ref[...] - rre_ref[...]
    y = 1.0 - aim_ref[...] - rim_ref[...]
    nrm = x * x + y * y
    gre = jnp.clip(x / nrm, -FEAT_CLAMP, FEAT_CLAMP)
    gim = jnp.clip(-y / nrm, -FEAT_CLAMP, FEAT_CLAMP)
    spec = gre * wo_ref[0:1, :] + gim * wo_ref[1:2, :] + bo_ref[...]
    out_ref[...] = ffn_ref[...] + sc_ref[0, 0] * spec


def _combine(ffn, v2, are, aim, rre, rim, Wo, bo, bk_scale):
    return pl.pallas_call(
        _combine_body,
        grid=(1,),
        in_specs=[
            pl.BlockSpec((N, D), lambda i: (0, 0)),
            pl.BlockSpec((N, 1), lambda i: (0, 0)),
            pl.BlockSpec((N, 1), lambda i: (0, 0)),
            pl.BlockSpec((N, 1), lambda i: (0, 0)),
            pl.BlockSpec((N, 1), lambda i: (0, 0)),
            pl.BlockSpec((N, 1), lambda i: (0, 0)),
            pl.BlockSpec((2, D), lambda i: (0, 0)),
            pl.BlockSpec((1, D), lambda i: (0, 0)),
            pl.BlockSpec(memory_space=pltpu.SMEM),
        ],
        out_specs=pl.BlockSpec((N, D), lambda i: (0, 0)),
        out_shape=jax.ShapeDtypeStruct((N, D), jnp.float32),
    )(ffn, v2, are, aim, rre, rim, Wo, bo.reshape(1, D),
      bk_scale.reshape(1, 1))


def kernel(x, Wr, br, W1, b1, W2, b2, Wv, bv, Wo, bo, bk_scale):
    x2 = x.reshape(N, D)
    ffn, v2 = _moe_ffn(x2, Wr, br, W1, b1, W2, b2, Wv, bv)
    v16 = v2.reshape(NCH, CH)
    v16rev = v16[::-1, ::-1]
    are, aim, bre, bim = _bk_scan(v16, v16rev)
    rre = bre[::-1, ::-1].reshape(N, 1)
    rim = bim[::-1, ::-1].reshape(N, 1)
    out = _combine(ffn, v2, are.reshape(N, 1), aim.reshape(N, 1),
                   rre, rim, Wo, bo, bk_scale)
    return out.reshape(x.shape)
